# Initial kernel scaffold; baseline (speedup 1.0000x reference)
import jax
import jax.numpy as jnp
from jax.experimental import pallas as pl


def kernel(x, edge_attr, edge_index, batch, enc_W1, enc_b1, ln_w, ln_b, enc_W2, enc_b2, g1_W, g1_as, g1_ad, g1_b, g2_W, g2_as, g2_ad, g2_b, g3_W, g3_as, g3_ad, g3_b, Wih, Whh, bih, bhh):
    raise NotImplementedError("write your pallas kernel here")


# speedup vs baseline: 5.5530x; 5.5530x over previous
import functools
import jax
import jax.numpy as jnp
from jax.experimental import pallas as pl
from jax.experimental.pallas import tpu as pltpu

_NB = 5000


def _encoder_body(x_ref, w1_ref, b1_ref, lnw_ref, lnb_ref, w2_ref, b2_ref,
                  o_ref):
    h = jnp.dot(x_ref[...], w1_ref[...].T,
                preferred_element_type=jnp.float32) + b1_ref[...]
    mu = jnp.mean(h, axis=-1, keepdims=True)
    var = jnp.mean((h - mu) ** 2, axis=-1, keepdims=True)
    h = (h - mu) * jax.lax.rsqrt(var + 1e-5) * lnw_ref[...] + lnb_ref[...]
    h = jnp.where(h >= 0, h, 0.1 * h)
    o_ref[...] = jnp.dot(h, w2_ref[...].T,
                         preferred_element_type=jnp.float32) + b2_ref[...]


def _encoder(x, w1, b1, lnw, lnb, w2, b2):
    n = x.shape[0]
    grid = n // _NB
    full = lambda s: pl.BlockSpec(s, lambda i: (0,) * len(s))
    return pl.pallas_call(
        _encoder_body,
        grid=(grid,),
        in_specs=[
            pl.BlockSpec((_NB, x.shape[1]), lambda i: (i, 0)),
            full(w1.shape), full((1, 128)), full((1, 128)), full((1, 128)),
            full(w2.shape), full((1, 256)),
        ],
        out_specs=pl.BlockSpec((_NB, 256), lambda i: (i, 0)),
        out_shape=jax.ShapeDtypeStruct((n, 256), jnp.float32),
    )(x, w1, b1.reshape(1, -1), lnw.reshape(1, -1), lnb.reshape(1, -1),
      w2, b2.reshape(1, -1))


def _transform_body(h_ref, w_ref, as_ref, ad_ref, xs_ref, s_ref, d_ref, *,
                    heads, ch):
    xs = jnp.dot(h_ref[...], w_ref[...].T,
                 preferred_element_type=jnp.float32)
    xs_ref[...] = xs
    x3 = xs.reshape(xs.shape[0], heads, ch)
    s_ref[...] = jnp.sum(x3 * as_ref[...], axis=-1)
    d_ref[...] = jnp.sum(x3 * ad_ref[...], axis=-1)


def _transform(h, w, att_s, att_d, heads, ch):
    n = h.shape[0]
    grid = n // _NB
    hc = heads * ch
    full = lambda s: pl.BlockSpec(s, lambda i: (0,) * len(s))
    return pl.pallas_call(
        functools.partial(_transform_body, heads=heads, ch=ch),
        grid=(grid,),
        in_specs=[
            pl.BlockSpec((_NB, h.shape[1]), lambda i: (i, 0)),
            full(w.shape), full(att_s.shape), full(att_d.shape),
        ],
        out_specs=[
            pl.BlockSpec((_NB, hc), lambda i: (i, 0)),
            pl.BlockSpec((_NB, heads), lambda i: (i, 0)),
            pl.BlockSpec((_NB, heads), lambda i: (i, 0)),
        ],
        out_shape=[
            jax.ShapeDtypeStruct((n, hc), jnp.float32),
            jax.ShapeDtypeStruct((n, heads), jnp.float32),
            jax.ShapeDtypeStruct((n, heads), jnp.float32),
        ],
    )(h, w, att_s, att_d)


def _norm_body(num_ref, den_ref, b_ref, o_ref, *, heads, ch):
    den = den_ref[...]
    den = jnp.repeat(den, ch, axis=-1)
    o_ref[...] = num_ref[...] / (den + 1e-16) + b_ref[...]


def _normalize(num, den, bias, heads, ch):
    n = num.shape[0]
    grid = n // _NB
    hc = heads * ch
    full = lambda s: pl.BlockSpec(s, lambda i: (0,) * len(s))
    return pl.pallas_call(
        functools.partial(_norm_body, heads=heads, ch=ch),
        grid=(grid,),
        in_specs=[
            pl.BlockSpec((_NB, hc), lambda i: (i, 0)),
            pl.BlockSpec((_NB, heads), lambda i: (i, 0)),
            full((1, hc)),
        ],
        out_specs=pl.BlockSpec((_NB, hc), lambda i: (i, 0)),
        out_shape=jax.ShapeDtypeStruct((n, hc), jnp.float32),
    )(num, den, bias.reshape(1, -1))


def _dotT(a, b):
    return jax.lax.dot_general(a, b, (((1,), (1,)), ((), ())),
                               preferred_element_type=jnp.float32)


def _dot0(a, b):
    return jax.lax.dot_general(a, b, (((0,), (0,)), ((), ())),
                               preferred_element_type=jnp.float32)


def _set2set_body(x_ref, bat_ref, wih_ref, whh_ref, bih_ref, bhh_ref, o_ref,
                  h_s, c_s, q_s, emax_s, acc_s):
    s = pl.program_id(0)
    p = pl.program_id(1)
    b = pl.program_id(2)
    nb = pl.num_programs(2)
    neg = jnp.float32(-1e30)

    @pl.when((p == 0) & (b == 0))
    def _step_head():
        @pl.when(s == 0)
        def _init():
            h_s[...] = jnp.zeros_like(h_s)
            c_s[...] = jnp.zeros_like(c_s)
            q_s[...] = jnp.zeros_like(q_s)

        g = (_dotT(q_s[...], wih_ref[...]) + bih_ref[...]
             + _dotT(h_s[...], whh_ref[...]) + bhh_ref[...])
        i = jax.nn.sigmoid(g[:, 0:64])
        f = jax.nn.sigmoid(g[:, 64:128])
        gg = jnp.tanh(g[:, 128:192])
        o = jax.nn.sigmoid(g[:, 192:256])
        c = f * c_s[...] + i * gg
        c_s[...] = c
        h_s[...] = o * jnp.tanh(c)
        emax_s[...] = jnp.full_like(emax_s, neg)

    blk = x_ref.shape[0]
    mask = (bat_ref[...] ==
            jax.lax.broadcasted_iota(jnp.int32, (blk, 16), 1))
    e = _dotT(x_ref[...], h_s[...])

    @pl.when(p == 0)
    def _max_pass():
        em = jnp.where(mask, e, neg)
        bm = jnp.max(em, axis=0, keepdims=True)
        emax_s[...] = jnp.maximum(emax_s[...], bm)

    @pl.when((p == 1) & (b == 0))
    def _fixup():
        m = emax_s[...]
        emax_s[...] = jnp.where(m > jnp.float32(-1e29), m, 0.0)
        acc_s[...] = jnp.zeros_like(acc_s)

    @pl.when(p == 1)
    def _acc_pass():
        ex = jnp.where(mask, jnp.exp(e - emax_s[...]), 0.0)
        xe = jnp.concatenate(
            [x_ref[...], jnp.ones((blk, 64), jnp.float32)], axis=-1)
        acc_s[...] += _dot0(ex, xe)

    @pl.when((p == 1) & (b == nb - 1))
    def _step_tail():
        acc = acc_s[...]
        r = acc[:, 0:64] / (acc[:, 64:65] + 1e-16)
        o_ref[...] = jnp.concatenate([h_s[...], r], axis=-1)
        q_s[...] = o_ref[...]


def _set2set(x, batch, wih, whh, bih, bhh):
    n = x.shape[0]
    nb = n // _NB
    full = lambda s: pl.BlockSpec(s, lambda st, p, b: (0,) * len(s))
    return pl.pallas_call(
        _set2set_body,
        grid=(5, 2, nb),
        in_specs=[
            pl.BlockSpec((_NB, 64), lambda st, p, b: (b, 0)),
            pl.BlockSpec((_NB, 1), lambda st, p, b: (b, 0)),
            full(wih.shape), full(whh.shape), full((1, 256)),
            full((1, 256)),
        ],
        out_specs=pl.BlockSpec((16, 128), lambda st, p, b: (0, 0)),
        out_shape=jax.ShapeDtypeStruct((16, 128), jnp.float32),
        scratch_shapes=[
            pltpu.VMEM((16, 64), jnp.float32),
            pltpu.VMEM((16, 64), jnp.float32),
            pltpu.VMEM((16, 128), jnp.float32),
            pltpu.VMEM((1, 16), jnp.float32),
            pltpu.VMEM((16, 128), jnp.float32),
        ],
    )(x, batch.reshape(n, 1), wih, whh, bih.reshape(1, -1),
      bhh.reshape(1, -1))


def kernel(x, edge_attr, edge_index, batch, enc_W1, enc_b1, ln_w, ln_b,
           enc_W2, enc_b2, g1_W, g1_as, g1_ad, g1_b, g2_W, g2_as, g2_ad,
           g2_b, g3_W, g3_as, g3_ad, g3_b, Wih, Whh, bih, bhh):
    n = x.shape[0]
    src0, dst0 = edge_index[0], edge_index[1]
    loop = jnp.arange(n, dtype=src0.dtype)
    src = jnp.concatenate([src0, loop])
    dst = jnp.concatenate([dst0, loop])
    valid = jnp.concatenate([src0 != dst0, jnp.ones((n,), bool)])

    h = _encoder(x, enc_W1, enc_b1, ln_w, ln_b, enc_W2, enc_b2)

    for (w, a_s, a_d, b, heads, ch) in (
            (g1_W, g1_as, g1_ad, g1_b, 2, 64),
            (g2_W, g2_as, g2_ad, g2_b, 2, 64),
            (g3_W, g3_as, g3_ad, g3_b, 1, 64)):
        xs, asv, adv = _transform(h, w, a_s, a_d, heads, ch)
        al = asv[src] + adv[dst]
        al = jnp.where(al >= 0, al, 0.2 * al)
        gmax = jnp.max(jnp.where(valid[:, None], al, -jnp.inf),
                       axis=0, keepdims=True)
        ex = jnp.where(valid[:, None], jnp.exp(al - gmax), 0.0)
        vals = jnp.repeat(ex, ch, axis=-1) * xs[src]
        num = jax.ops.segment_sum(vals, dst, num_segments=n)
        den = jax.ops.segment_sum(ex, dst, num_segments=n)
        h = _normalize(num, den, b, heads, ch)

    return _set2set(h, batch, Wih, Whh, bih, bhh)


# speedup vs baseline: 25.9590x; 4.6748x over previous
import functools
import jax
import jax.numpy as jnp
from jax.experimental import pallas as pl
from jax.experimental.pallas import tpu as pltpu
from jax.experimental.pallas import tpu_sc as plsc

_NB = 5000
_N = 50000
_NPAD = 50176
_DEN = 100352
_W = 128


def _encoder_body(x_ref, w1_ref, b1_ref, lnw_ref, lnb_ref, w2_ref, b2_ref,
                  o_ref):
    h = jnp.dot(x_ref[...], w1_ref[...].T,
                preferred_element_type=jnp.float32) + b1_ref[...]
    mu = jnp.mean(h, axis=-1, keepdims=True)
    var = jnp.mean((h - mu) ** 2, axis=-1, keepdims=True)
    h = (h - mu) * jax.lax.rsqrt(var + 1e-5) * lnw_ref[...] + lnb_ref[...]
    h = jnp.where(h >= 0, h, 0.1 * h)
    o_ref[...] = jnp.dot(h, w2_ref[...].T,
                         preferred_element_type=jnp.float32) + b2_ref[...]


def _encoder(x, w1, b1, lnw, lnb, w2, b2):
    n = x.shape[0]
    grid = n // _NB
    full = lambda s: pl.BlockSpec(s, lambda i: (0,) * len(s))
    return pl.pallas_call(
        _encoder_body,
        grid=(grid,),
        in_specs=[
            pl.BlockSpec((_NB, x.shape[1]), lambda i: (i, 0)),
            full(w1.shape), full((1, 128)), full((1, 128)), full((1, 128)),
            full(w2.shape), full((1, 256)),
        ],
        out_specs=pl.BlockSpec((_NB, 256), lambda i: (i, 0)),
        out_shape=jax.ShapeDtypeStruct((n, 256), jnp.float32),
    )(x, w1, b1.reshape(1, -1), lnw.reshape(1, -1), lnb.reshape(1, -1),
      w2, b2.reshape(1, -1))


def _transform_body(h_ref, w_ref, as_ref, ad_ref, *out_refs, heads, ch,
                    nslice):
    xs_refs = out_refs[:nslice]
    att_ref, gm_ref = out_refs[nslice], out_refs[nslice + 1]
    i = pl.program_id(0)
    xs = jnp.dot(h_ref[...], w_ref[...].T,
                 preferred_element_type=jnp.float32)
    for sl in range(nslice):
        xs_refs[sl][...] = xs[:, 32 * sl:32 * sl + 32]
    x3 = xs.reshape(xs.shape[0], heads, ch)
    a_s = jnp.sum(x3 * as_ref[...], axis=-1)
    a_d = jnp.sum(x3 * ad_ref[...], axis=-1)
    nb = xs.shape[0]
    att_ref[...] = jnp.concatenate(
        [a_s, a_d, jnp.zeros((nb, 16 - 2 * heads), jnp.float32)], axis=-1)
    bm = jnp.concatenate(
        [jnp.max(a_s, axis=0, keepdims=True),
         jnp.max(a_d, axis=0, keepdims=True),
         jnp.full((1, 128 - 2 * heads), -1e30, jnp.float32)], axis=-1)

    @pl.when(i == 0)
    def _init():
        gm_ref[...] = jnp.full_like(gm_ref, -1e30)

    gm_ref[...] = jnp.maximum(gm_ref[...], bm)


def _transform(h, w, att_s, att_d, heads, ch):
    n = h.shape[0]
    grid = n // _NB
    nslice = heads * ch // 32
    full = lambda s: pl.BlockSpec(s, lambda i: (0,) * len(s))
    outs = pl.pallas_call(
        functools.partial(_transform_body, heads=heads, ch=ch,
                          nslice=nslice),
        grid=(grid,),
        in_specs=[
            pl.BlockSpec((_NB, h.shape[1]), lambda i: (i, 0)),
            full(w.shape), full(att_s.shape), full(att_d.shape),
        ],
        out_specs=(
            [pl.BlockSpec((_NB, 32), lambda i: (i, 0))] * nslice
            + [pl.BlockSpec((_NB, 16), lambda i: (i, 0)),
               pl.BlockSpec((1, 128), lambda i: (0, 0))]),
        out_shape=(
            [jax.ShapeDtypeStruct((n, 32), jnp.float32)] * nslice
            + [jax.ShapeDtypeStruct((n, 16), jnp.float32),
               jax.ShapeDtypeStruct((1, 128), jnp.float32)]),
    )(h, w, att_s, att_d)
    return outs[:nslice], outs[nslice], outs[nslice + 1]


def _norm_body(np_ref, den_ref, b_ref, o_ref, *, heads, nslice):
    d = den_ref[...][:, 0:2] + den_ref[...][:, 2:4]
    npv = np_ref[...]
    cols = []
    for sl in range(nslice):
        num = (npv[:, 32 * sl:32 * sl + 32]
               + npv[:, 32 * (nslice + sl):32 * (nslice + sl) + 32])
        dc = d[:, sl // 2:sl // 2 + 1]
        cols.append(num / (dc + 1e-16))
    o_ref[...] = jnp.concatenate(cols, axis=-1) + b_ref[...]


def _normalize(num_parts, den2, bias, heads, nslice):
    hc = 32 * nslice
    grid = _N // _NB
    np2 = jnp.transpose(num_parts[:, :_N], (1, 0, 2)).reshape(
        _N, 2 * nslice * 32)
    den3 = jnp.transpose(den2.reshape(2, _DEN // 2, 2)[:, :_N],
                         (1, 0, 2)).reshape(_N, 4)
    full = lambda s: pl.BlockSpec(s, lambda i: (0,) * len(s))
    return pl.pallas_call(
        functools.partial(_norm_body, heads=heads, nslice=nslice),
        grid=(grid,),
        in_specs=[
            pl.BlockSpec((_NB, 2 * nslice * 32), lambda i: (i, 0)),
            pl.BlockSpec((_NB, 4), lambda i: (i, 0)),
            full((1, hc)),
        ],
        out_specs=pl.BlockSpec((_NB, hc), lambda i: (i, 0)),
        out_shape=jax.ShapeDtypeStruct((_N, hc), jnp.float32),
    )(np2, den3, bias.reshape(1, -1))


def _bc16(v, j):
    idx = jnp.full((16, 1), j, jnp.int32)
    dn = jax.lax.GatherDimensionNumbers(
        offset_dims=(), collapsed_slice_dims=(0,), start_index_map=(0,))
    return jax.lax.gather(
        v, idx, dn, (1,),
        mode=jax.lax.GatherScatterMode.PROMISE_IN_BOUNDS)


def _make_edge_ex(heads, e_orig, e_live, e2p):
    nw = e2p // (32 * _W)
    mesh = plsc.VectorSubcoreMesh(core_axis_name="c", subcore_axis_name="s")

    def body(*refs):
        as_refs = refs[:heads]
        ad_refs = refs[heads:2 * heads]
        src_ref, dst_ref, gmax_ref, zden_ref = refs[2 * heads:2 * heads + 4]
        ext_ref, den2_ref = refs[2 * heads + 4:2 * heads + 6]
        sc = refs[2 * heads + 6:]
        srcv, dstv = sc[0], sc[1]
        asb = sc[2:2 + heads]
        adb = sc[2 + heads:2 + 2 * heads]
        exst, idxb, valb, gmaxv, den_sh = sc[2 + 2 * heads:7 + 2 * heads]
        sems = sc[7 + 2 * heads:]
        c = jax.lax.axis_index("c")
        s = jax.lax.axis_index("s")
        lo0 = (c * 16 + s) * (nw * _W)
        pltpu.sync_copy(gmax_ref, gmaxv)
        pltpu.sync_copy(zden_ref, den_sh.at[pl.ds(s * 6272, 6272)])
        plsc.subcore_barrier()
        gv = gmaxv[...]
        iota = jax.lax.iota(jnp.int32, 16)

        def window(w, carry):
            off = lo0 + w * _W
            pltpu.sync_copy(src_ref.at[pl.ds(off, _W)], srcv)
            pltpu.sync_copy(dst_ref.at[pl.ds(off, _W)], dstv)
            cps = []
            for h in range(heads):
                cps.append(pltpu.async_copy(
                    as_refs[h].at[srcv], asb[h], sems[2 * h]))
                cps.append(pltpu.async_copy(
                    ad_refs[h].at[dstv], adb[h], sems[2 * h + 1]))
            for cp in cps:
                cp.wait()
            for h in range(heads):
                bc = _bc16(gv, h)
                for g in range(8):
                    sv = srcv[pl.ds(g * 16, 16)]
                    dv = dstv[pl.ds(g * 16, 16)]
                    rid = g * 16 + iota
                    al = asb[h][pl.ds(g * 16, 16)] + adb[h][pl.ds(g * 16, 16)]
                    al = jnp.where(al >= 0, al, 0.2 * al)
                    eid = off + rid
                    inval = (sv == dv) & ((eid < e_orig) | (eid >= e_live))
                    ex = jnp.where(inval, 0.0, jnp.exp(al - bc))
                    exst[h, pl.ds(g * 16, 16)] = ex
                    valb[pl.ds(g * 16, 16)] = ex
                    idxb[pl.ds(g * 16, 16)] = dv * 2 + h
                pltpu.sync_copy(exst.at[h], ext_ref.at[h, pl.ds(off, _W)])
                pltpu.sync_copy(valb, den_sh.at[idxb], add=True)
            return carry

        jax.lax.fori_loop(0, nw, window, 0)
        plsc.subcore_barrier()
        pltpu.sync_copy(den_sh.at[pl.ds(s * 6272, 6272)],
                        den2_ref.at[c, pl.ds(s * 6272, 6272)])

    return pl.kernel(
        body,
        out_type=[jax.ShapeDtypeStruct((2, e2p), jnp.float32),
                  jax.ShapeDtypeStruct((2, _DEN), jnp.float32)],
        mesh=mesh,
        scratch_types=(
            [pltpu.VMEM((_W,), jnp.int32),
             pltpu.VMEM((_W,), jnp.int32)]
            + [pltpu.VMEM((_W,), jnp.float32)] * (2 * heads)
            + [pltpu.VMEM((2, _W), jnp.float32),
               pltpu.VMEM((_W,), jnp.int32),
               pltpu.VMEM((_W,), jnp.float32),
               pltpu.VMEM((16,), jnp.float32),
               pltpu.VMEM_SHARED((_DEN,), jnp.float32)]
            + [pltpu.SemaphoreType.DMA] * (2 * heads)),
    )


def _make_edge_agg(nslice, e2p):
    nw = e2p // (32 * _W)
    mesh = plsc.VectorSubcoreMesh(core_axis_name="c", subcore_axis_name="s")

    def body(*refs):
        xs_refs = refs[:nslice]
        src_ref, dst_ref, ext_ref, zacc_ref = refs[nslice:nslice + 4]
        out_ref = refs[nslice + 4]
        srcw, dstw, exw, rows, scaled, acc_sh, sem = refs[nslice + 5:]
        c = jax.lax.axis_index("c")
        s = jax.lax.axis_index("s")
        lo0 = (c * 16 + s) * (nw * _W)
        for sl in range(nslice):
            h = sl // 2
            for k in range(4):
                pltpu.sync_copy(
                    zacc_ref, acc_sh.at[pl.ds(s * 3136 + k * 784, 784)])
            plsc.subcore_barrier()

            def window(w, carry):
                off = lo0 + w * _W
                pltpu.sync_copy(src_ref.at[pl.ds(off, _W)], srcw)
                pltpu.sync_copy(dst_ref.at[pl.ds(off, _W)], dstw)
                pltpu.sync_copy(ext_ref.at[h, pl.ds(off, _W)], exw)
                pltpu.async_copy(xs_refs[sl].at[srcw], rows, sem).wait()
                for g in range(8):
                    exv = exw[pl.ds(g * 16, 16)]
                    for j in range(16):
                        e = g * 16 + j
                        bc = _bc16(exv, j)
                        scaled[e, pl.ds(0, 16)] = rows[e, pl.ds(0, 16)] * bc
                        scaled[e, pl.ds(16, 16)] = (rows[e, pl.ds(16, 16)]
                                                    * bc)
                pltpu.sync_copy(scaled, acc_sh.at[dstw], add=True)
                return carry

            jax.lax.fori_loop(0, nw, window, 0)
            plsc.subcore_barrier()
            pltpu.sync_copy(
                acc_sh.at[pl.ds(s * 3136, 3136)],
                out_ref.at[c * nslice + sl, pl.ds(s * 3136, 3136)])
            plsc.subcore_barrier()

    return pl.kernel(
        body,
        out_type=jax.ShapeDtypeStruct((2 * nslice, _NPAD, 32), jnp.float32),
        mesh=mesh,
        compiler_params=pltpu.CompilerParams(use_tc_tiling_on_sc=False),
        scratch_types=[
            pltpu.VMEM((_W,), jnp.int32),
            pltpu.VMEM((_W,), jnp.int32),
            pltpu.VMEM((_W,), jnp.float32),
            pltpu.VMEM((_W, 32), jnp.float32),
            pltpu.VMEM((_W, 32), jnp.float32),
            pltpu.VMEM_SHARED((_NPAD, 32), jnp.float32),
            pltpu.SemaphoreType.DMA,
        ],
    )


def _dotT(a, b):
    return jax.lax.dot_general(a, b, (((1,), (1,)), ((), ())),
                               preferred_element_type=jnp.float32)


def _dot0(a, b):
    return jax.lax.dot_general(a, b, (((0,), (0,)), ((), ())),
                               preferred_element_type=jnp.float32)


def _set2set_body(x_ref, bat_ref, wih_ref, whh_ref, bih_ref, bhh_ref, o_ref,
                  h_s, c_s, q_s, emax_s, acc_s):
    s = pl.program_id(0)
    p = pl.program_id(1)
    b = pl.program_id(2)
    nb = pl.num_programs(2)
    neg = jnp.float32(-1e30)

    @pl.when((p == 0) & (b == 0))
    def _step_head():
        @pl.when(s == 0)
        def _init():
            h_s[...] = jnp.zeros_like(h_s)
            c_s[...] = jnp.zeros_like(c_s)
            q_s[...] = jnp.zeros_like(q_s)

        g = (_dotT(q_s[...], wih_ref[...]) + bih_ref[...]
             + _dotT(h_s[...], whh_ref[...]) + bhh_ref[...])
        i = jax.nn.sigmoid(g[:, 0:64])
        f = jax.nn.sigmoid(g[:, 64:128])
        gg = jnp.tanh(g[:, 128:192])
        o = jax.nn.sigmoid(g[:, 192:256])
        c = f * c_s[...] + i * gg
        c_s[...] = c
        h_s[...] = o * jnp.tanh(c)
        emax_s[...] = jnp.full_like(emax_s, neg)

    blk = x_ref.shape[0]
    mask = (bat_ref[...] ==
            jax.lax.broadcasted_iota(jnp.int32, (blk, 16), 1))
    e = _dotT(x_ref[...], h_s[...])

    @pl.when(p == 0)
    def _max_pass():
        em = jnp.where(mask, e, neg)
        bm = jnp.max(em, axis=0, keepdims=True)
        emax_s[...] = jnp.maximum(emax_s[...], bm)

    @pl.when((p == 1) & (b == 0))
    def _fixup():
        m = emax_s[...]
        emax_s[...] = jnp.where(m > jnp.float32(-1e29), m, 0.0)
        acc_s[...] = jnp.zeros_like(acc_s)

    @pl.when(p == 1)
    def _acc_pass():
        ex = jnp.where(mask, jnp.exp(e - emax_s[...]), 0.0)
        xe = jnp.concatenate(
            [x_ref[...], jnp.ones((blk, 64), jnp.float32)], axis=-1)
        acc_s[...] += _dot0(ex, xe)

    @pl.when((p == 1) & (b == nb - 1))
    def _step_tail():
        acc = acc_s[...]
        r = acc[:, 0:64] / (acc[:, 64:65] + 1e-16)
        o_ref[...] = jnp.concatenate([h_s[...], r], axis=-1)
        q_s[...] = o_ref[...]


def _set2set(x, batch, wih, whh, bih, bhh):
    n = x.shape[0]
    nb = n // _NB
    full = lambda s: pl.BlockSpec(s, lambda st, p, b: (0,) * len(s))
    return pl.pallas_call(
        _set2set_body,
        grid=(5, 2, nb),
        in_specs=[
            pl.BlockSpec((_NB, 64), lambda st, p, b: (b, 0)),
            pl.BlockSpec((_NB, 1), lambda st, p, b: (b, 0)),
            full(wih.shape), full(whh.shape), full((1, 256)),
            full((1, 256)),
        ],
        out_specs=pl.BlockSpec((16, 128), lambda st, p, b: (0, 0)),
        out_shape=jax.ShapeDtypeStruct((16, 128), jnp.float32),
        scratch_shapes=[
            pltpu.VMEM((16, 64), jnp.float32),
            pltpu.VMEM((16, 64), jnp.float32),
            pltpu.VMEM((16, 128), jnp.float32),
            pltpu.VMEM((1, 16), jnp.float32),
            pltpu.VMEM((16, 128), jnp.float32),
        ],
    )(x, batch.reshape(n, 1), wih, whh, bih.reshape(1, -1),
      bhh.reshape(1, -1))


def kernel(x, edge_attr, edge_index, batch, enc_W1, enc_b1, ln_w, ln_b,
           enc_W2, enc_b2, g1_W, g1_as, g1_ad, g1_b, g2_W, g2_as, g2_ad,
           g2_b, g3_W, g3_as, g3_ad, g3_b, Wih, Whh, bih, bhh):
    n = x.shape[0]
    e_orig = edge_index.shape[1]
    e_live = e_orig + n
    e2p = ((e_live + 4095) // 4096) * 4096
    src0, dst0 = edge_index[0], edge_index[1]
    loop = jnp.arange(n, dtype=src0.dtype)
    padz = jnp.zeros((e2p - e_live,), src0.dtype)
    src = jnp.concatenate([src0, loop, padz])
    dst = jnp.concatenate([dst0, loop, padz])
    zden = jnp.zeros((6272,), jnp.float32)
    zacc = jnp.zeros((784, 32), jnp.float32)

    h = _encoder(x, enc_W1, enc_b1, ln_w, ln_b, enc_W2, enc_b2)

    for (w, a_s, a_d, b, heads, ch) in (
            (g1_W, g1_as, g1_ad, g1_b, 2, 64),
            (g2_W, g2_as, g2_ad, g2_b, 2, 64),
            (g3_W, g3_as, g3_ad, g3_b, 1, 64)):
        nslice = heads * ch // 32
        xs_list, att16, gm = _transform(h, w, a_s, a_d, heads, ch)
        gmax16 = jnp.concatenate(
            [gm[0, 0:heads] + gm[0, heads:2 * heads],
             jnp.zeros((16 - heads,), jnp.float32)])
        a_cols = ([att16[:, h] for h in range(heads)]
                  + [att16[:, heads + h] for h in range(heads)])
        ex_t, den2 = _make_edge_ex(heads, e_orig, e_live, e2p)(
            *a_cols, src, dst, gmax16, zden)
        num_parts = _make_edge_agg(nslice, e2p)(
            *xs_list, src, dst, ex_t, zacc)
        h = _normalize(num_parts, den2, b, heads, nslice)

    return _set2set(h, batch, Wih, Whh, bih, bhh)


# speedup vs baseline: 39.1904x; 1.5097x over previous
import functools
import jax
import jax.numpy as jnp
from jax.experimental import pallas as pl
from jax.experimental.pallas import tpu as pltpu
from jax.experimental.pallas import tpu_sc as plsc

_NB = 5000
_N = 50000
_NPAD = 50176
_DEN = 100352
_W = 128


def _encoder_body(x_ref, w1_ref, b1_ref, lnw_ref, lnb_ref, w2_ref, b2_ref,
                  o_ref):
    h = jnp.dot(x_ref[...], w1_ref[...].T,
                preferred_element_type=jnp.float32) + b1_ref[...]
    mu = jnp.mean(h, axis=-1, keepdims=True)
    var = jnp.mean((h - mu) ** 2, axis=-1, keepdims=True)
    h = (h - mu) * jax.lax.rsqrt(var + 1e-5) * lnw_ref[...] + lnb_ref[...]
    h = jnp.where(h >= 0, h, 0.1 * h)
    o_ref[...] = jnp.dot(h, w2_ref[...].T,
                         preferred_element_type=jnp.float32) + b2_ref[...]


def _encoder(x, w1, b1, lnw, lnb, w2, b2):
    n = x.shape[0]
    grid = n // _NB
    full = lambda s: pl.BlockSpec(s, lambda i: (0,) * len(s))
    return pl.pallas_call(
        _encoder_body,
        grid=(grid,),
        in_specs=[
            pl.BlockSpec((_NB, x.shape[1]), lambda i: (i, 0)),
            full(w1.shape), full((1, 128)), full((1, 128)), full((1, 128)),
            full(w2.shape), full((1, 256)),
        ],
        out_specs=pl.BlockSpec((_NB, 256), lambda i: (i, 0)),
        out_shape=jax.ShapeDtypeStruct((n, 256), jnp.float32),
    )(x, w1, b1.reshape(1, -1), lnw.reshape(1, -1), lnb.reshape(1, -1),
      w2, b2.reshape(1, -1))


def _transform_body(h_ref, w_ref, as_ref, ad_ref, *out_refs, heads, ch,
                    nslice):
    xs_refs = out_refs[:nslice]
    att_ref, gm_ref = out_refs[nslice], out_refs[nslice + 1]
    i = pl.program_id(0)
    xs = jnp.dot(h_ref[...], w_ref[...].T,
                 preferred_element_type=jnp.float32)
    for sl in range(nslice):
        xs_refs[sl][...] = xs[:, 32 * sl:32 * sl + 32]
    x3 = xs.reshape(xs.shape[0], heads, ch)
    a_s = jnp.sum(x3 * as_ref[...], axis=-1)
    a_d = jnp.sum(x3 * ad_ref[...], axis=-1)
    nb = xs.shape[0]
    att_ref[...] = jnp.concatenate(
        [a_s, a_d, jnp.zeros((nb, 16 - 2 * heads), jnp.float32)], axis=-1)
    bm = jnp.concatenate(
        [jnp.max(a_s, axis=0, keepdims=True),
         jnp.max(a_d, axis=0, keepdims=True),
         jnp.full((1, 128 - 2 * heads), -1e30, jnp.float32)], axis=-1)

    @pl.when(i == 0)
    def _init():
        gm_ref[...] = jnp.full_like(gm_ref, -1e30)

    gm_ref[...] = jnp.maximum(gm_ref[...], bm)


def _transform(h, w, att_s, att_d, heads, ch):
    n = h.shape[0]
    grid = n // _NB
    nslice = heads * ch // 32
    full = lambda s: pl.BlockSpec(s, lambda i: (0,) * len(s))
    outs = pl.pallas_call(
        functools.partial(_transform_body, heads=heads, ch=ch,
                          nslice=nslice),
        grid=(grid,),
        in_specs=[
            pl.BlockSpec((_NB, h.shape[1]), lambda i: (i, 0)),
            full(w.shape), full(att_s.shape), full(att_d.shape),
        ],
        out_specs=(
            [pl.BlockSpec((_NB, 32), lambda i: (i, 0))] * nslice
            + [pl.BlockSpec((_NB, 16), lambda i: (i, 0)),
               pl.BlockSpec((1, 128), lambda i: (0, 0))]),
        out_shape=(
            [jax.ShapeDtypeStruct((n, 32), jnp.float32)] * nslice
            + [jax.ShapeDtypeStruct((n, 16), jnp.float32),
               jax.ShapeDtypeStruct((1, 128), jnp.float32)]),
    )(h, w, att_s, att_d)
    return outs[:nslice], outs[nslice], outs[nslice + 1]


def _norm_body(np_ref, den_ref, b_ref, o_ref, *, heads, nslice):
    d = den_ref[...][:, 0:2] + den_ref[...][:, 2:4]
    npv = np_ref[...]
    cols = []
    for sl in range(nslice):
        num = (npv[:, 32 * sl:32 * sl + 32]
               + npv[:, 32 * (nslice + sl):32 * (nslice + sl) + 32])
        dc = d[:, sl // 2:sl // 2 + 1]
        cols.append(num / (dc + 1e-16))
    o_ref[...] = jnp.concatenate(cols, axis=-1) + b_ref[...]


def _normalize(num_parts, den2, bias, heads, nslice):
    hc = 32 * nslice
    grid = _N // _NB
    np2 = jnp.transpose(num_parts[:, :_N], (1, 0, 2)).reshape(
        _N, 2 * nslice * 32)
    den3 = jnp.transpose(den2.reshape(2, _DEN // 2, 2)[:, :_N],
                         (1, 0, 2)).reshape(_N, 4)
    full = lambda s: pl.BlockSpec(s, lambda i: (0,) * len(s))
    return pl.pallas_call(
        functools.partial(_norm_body, heads=heads, nslice=nslice),
        grid=(grid,),
        in_specs=[
            pl.BlockSpec((_NB, 2 * nslice * 32), lambda i: (i, 0)),
            pl.BlockSpec((_NB, 4), lambda i: (i, 0)),
            full((1, hc)),
        ],
        out_specs=pl.BlockSpec((_NB, hc), lambda i: (i, 0)),
        out_shape=jax.ShapeDtypeStruct((_N, hc), jnp.float32),
    )(np2, den3, bias.reshape(1, -1))


def _bc16(v, j):
    idx = jnp.full((16, 1), j, jnp.int32)
    dn = jax.lax.GatherDimensionNumbers(
        offset_dims=(), collapsed_slice_dims=(0,), start_index_map=(0,))
    return jax.lax.gather(
        v, idx, dn, (1,),
        mode=jax.lax.GatherScatterMode.PROMISE_IN_BOUNDS)


def _make_edge_ex(heads, e_orig, e_live, e2p):
    nw = e2p // (32 * _W)
    mesh = plsc.VectorSubcoreMesh(core_axis_name="c", subcore_axis_name="s")

    def body(*refs):
        as_refs = refs[:heads]
        ad_refs = refs[heads:2 * heads]
        src_ref, dst_ref, gmax_ref, zden_ref = refs[2 * heads:2 * heads + 4]
        ext_ref, den2_ref = refs[2 * heads + 4:2 * heads + 6]
        sc = refs[2 * heads + 6:]
        srcv, dstv = sc[0], sc[1]
        asb = sc[2:2 + heads]
        adb = sc[2 + heads:2 + 2 * heads]
        exst, idxb, valb, gmaxv, den_sh = sc[2 + 2 * heads:7 + 2 * heads]
        sems = sc[7 + 2 * heads:]
        c = jax.lax.axis_index("c")
        s = jax.lax.axis_index("s")
        lo0 = (c * 16 + s) * (nw * _W)
        pltpu.sync_copy(gmax_ref, gmaxv)
        pltpu.sync_copy(zden_ref, den_sh.at[pl.ds(s * 6272, 6272)])
        plsc.subcore_barrier()
        gv = gmaxv[...]
        iota = jax.lax.iota(jnp.int32, 16)

        def window(w, carry):
            off = lo0 + w * _W
            pltpu.sync_copy(src_ref.at[pl.ds(off, _W)], srcv)
            pltpu.sync_copy(dst_ref.at[pl.ds(off, _W)], dstv)
            cps = []
            for h in range(heads):
                cps.append(pltpu.async_copy(
                    as_refs[h].at[srcv], asb[h], sems[2 * h]))
                cps.append(pltpu.async_copy(
                    ad_refs[h].at[dstv], adb[h], sems[2 * h + 1]))
            for cp in cps:
                cp.wait()
            for h in range(heads):
                bc = _bc16(gv, h)
                for g in range(8):
                    sv = srcv[pl.ds(g * 16, 16)]
                    dv = dstv[pl.ds(g * 16, 16)]
                    rid = g * 16 + iota
                    al = asb[h][pl.ds(g * 16, 16)] + adb[h][pl.ds(g * 16, 16)]
                    al = jnp.where(al >= 0, al, 0.2 * al)
                    eid = off + rid
                    inval = (sv == dv) & ((eid < e_orig) | (eid >= e_live))
                    ex = jnp.where(inval, 0.0, jnp.exp(al - bc))
                    exst[h, pl.ds(g * 16, 16)] = ex
                    valb[pl.ds(g * 16, 16)] = ex
                    idxb[pl.ds(g * 16, 16)] = dv * 2 + h
                pltpu.sync_copy(exst.at[h], ext_ref.at[h, pl.ds(off, _W)])
                pltpu.sync_copy(valb, den_sh.at[idxb], add=True)
            return carry

        jax.lax.fori_loop(0, nw, window, 0)
        plsc.subcore_barrier()
        pltpu.sync_copy(den_sh.at[pl.ds(s * 6272, 6272)],
                        den2_ref.at[c, pl.ds(s * 6272, 6272)])

    return pl.kernel(
        body,
        out_type=[jax.ShapeDtypeStruct((2, e2p), jnp.float32),
                  jax.ShapeDtypeStruct((2, _DEN), jnp.float32)],
        mesh=mesh,
        scratch_types=(
            [pltpu.VMEM((_W,), jnp.int32),
             pltpu.VMEM((_W,), jnp.int32)]
            + [pltpu.VMEM((_W,), jnp.float32)] * (2 * heads)
            + [pltpu.VMEM((2, _W), jnp.float32),
               pltpu.VMEM((_W,), jnp.int32),
               pltpu.VMEM((_W,), jnp.float32),
               pltpu.VMEM((16,), jnp.float32),
               pltpu.VMEM_SHARED((_DEN,), jnp.float32)]
            + [pltpu.SemaphoreType.DMA] * (2 * heads)),
    )


def _make_edge_agg(nslice, e2p):
    nw = e2p // (32 * _W)
    mesh = plsc.VectorSubcoreMesh(core_axis_name="c", subcore_axis_name="s")

    def body(*refs):
        xs_refs = refs[:nslice]
        src_ref, dst_ref, ext_ref, zacc_ref = refs[nslice:nslice + 4]
        out_ref = refs[nslice + 4]
        sc = refs[nslice + 5:]
        srcw, dstw, exw, rows, scaled = (sc[0:2], sc[2:4], sc[4:6],
                                         sc[6:8], sc[8:10])
        acc_sh = sc[10]
        sidx = sc[11:17]
        sgat = sc[17:19]
        ssc = sc[19:21]
        c = jax.lax.axis_index("c")
        s = jax.lax.axis_index("s")
        lo0 = (c * 16 + s) * (nw * _W)

        def compute(p):
            for g in range(8):
                exv = exw[p][pl.ds(g * 16, 16)]
                for j in range(16):
                    e = g * 16 + j
                    bc = _bc16(exv, j)
                    scaled[p][e, pl.ds(0, 16)] = (rows[p][e, pl.ds(0, 16)]
                                                  * bc)
                    scaled[p][e, pl.ds(16, 16)] = (rows[p][e, pl.ds(16, 16)]
                                                   * bc)

        for sl in range(nslice):
            h = sl // 2
            for k in range(4):
                pltpu.sync_copy(
                    zacc_ref, acc_sh.at[pl.ds(s * 3136 + k * 784, 784)])
            plsc.subcore_barrier()

            def pair(wp, carry):
                offs = [lo0 + (2 * wp + p) * _W for p in range(2)]
                cps = []
                for p in range(2):
                    cps.append([
                        pltpu.async_copy(src_ref.at[pl.ds(offs[p], _W)],
                                         srcw[p], sidx[3 * p]),
                        pltpu.async_copy(dst_ref.at[pl.ds(offs[p], _W)],
                                         dstw[p], sidx[3 * p + 1]),
                        pltpu.async_copy(ext_ref.at[h, pl.ds(offs[p], _W)],
                                         exw[p], sidx[3 * p + 2])])
                gth = []
                for p in range(2):
                    for cp in cps[p]:
                        cp.wait()
                    gth.append(pltpu.async_copy(
                        xs_refs[sl].at[srcw[p]], rows[p], sgat[p]))
                gth[0].wait()
                compute(0)
                sc0 = pltpu.async_copy(scaled[0], acc_sh.at[dstw[0]],
                                       ssc[0], add=True)
                gth[1].wait()
                compute(1)
                sc1 = pltpu.async_copy(scaled[1], acc_sh.at[dstw[1]],
                                       ssc[1], add=True)
                sc0.wait()
                sc1.wait()
                return carry

            jax.lax.fori_loop(0, nw // 2, pair, 0)
            plsc.subcore_barrier()
            pltpu.sync_copy(
                acc_sh.at[pl.ds(s * 3136, 3136)],
                out_ref.at[c * nslice + sl, pl.ds(s * 3136, 3136)])
            plsc.subcore_barrier()

    return pl.kernel(
        body,
        out_type=jax.ShapeDtypeStruct((2 * nslice, _NPAD, 32), jnp.float32),
        mesh=mesh,
        compiler_params=pltpu.CompilerParams(use_tc_tiling_on_sc=False),
        scratch_types=(
            [pltpu.VMEM((_W,), jnp.int32)] * 2
            + [pltpu.VMEM((_W,), jnp.int32)] * 2
            + [pltpu.VMEM((_W,), jnp.float32)] * 2
            + [pltpu.VMEM((_W, 32), jnp.float32)] * 2
            + [pltpu.VMEM((_W, 32), jnp.float32)] * 2
            + [pltpu.VMEM_SHARED((_NPAD, 32), jnp.float32)]
            + [pltpu.SemaphoreType.DMA] * 10),
    )


def _dotT(a, b):
    return jax.lax.dot_general(a, b, (((1,), (1,)), ((), ())),
                               preferred_element_type=jnp.float32)


def _dot0(a, b):
    return jax.lax.dot_general(a, b, (((0,), (0,)), ((), ())),
                               preferred_element_type=jnp.float32)


def _set2set_body(x_ref, bat_ref, wih_ref, whh_ref, bih_ref, bhh_ref, o_ref,
                  h_s, c_s, q_s, emax_s, acc_s):
    s = pl.program_id(0)
    p = pl.program_id(1)
    b = pl.program_id(2)
    nb = pl.num_programs(2)
    neg = jnp.float32(-1e30)

    @pl.when((p == 0) & (b == 0))
    def _step_head():
        @pl.when(s == 0)
        def _init():
            h_s[...] = jnp.zeros_like(h_s)
            c_s[...] = jnp.zeros_like(c_s)
            q_s[...] = jnp.zeros_like(q_s)

        g = (_dotT(q_s[...], wih_ref[...]) + bih_ref[...]
             + _dotT(h_s[...], whh_ref[...]) + bhh_ref[...])
        i = jax.nn.sigmoid(g[:, 0:64])
        f = jax.nn.sigmoid(g[:, 64:128])
        gg = jnp.tanh(g[:, 128:192])
        o = jax.nn.sigmoid(g[:, 192:256])
        c = f * c_s[...] + i * gg
        c_s[...] = c
        h_s[...] = o * jnp.tanh(c)
        emax_s[...] = jnp.full_like(emax_s, neg)

    blk = x_ref.shape[0]
    mask = (bat_ref[...] ==
            jax.lax.broadcasted_iota(jnp.int32, (blk, 16), 1))
    e = _dotT(x_ref[...], h_s[...])

    @pl.when(p == 0)
    def _max_pass():
        em = jnp.where(mask, e, neg)
        bm = jnp.max(em, axis=0, keepdims=True)
        emax_s[...] = jnp.maximum(emax_s[...], bm)

    @pl.when((p == 1) & (b == 0))
    def _fixup():
        m = emax_s[...]
        emax_s[...] = jnp.where(m > jnp.float32(-1e29), m, 0.0)
        acc_s[...] = jnp.zeros_like(acc_s)

    @pl.when(p == 1)
    def _acc_pass():
        ex = jnp.where(mask, jnp.exp(e - emax_s[...]), 0.0)
        xe = jnp.concatenate(
            [x_ref[...], jnp.ones((blk, 64), jnp.float32)], axis=-1)
        acc_s[...] += _dot0(ex, xe)

    @pl.when((p == 1) & (b == nb - 1))
    def _step_tail():
        acc = acc_s[...]
        r = acc[:, 0:64] / (acc[:, 64:65] + 1e-16)
        o_ref[...] = jnp.concatenate([h_s[...], r], axis=-1)
        q_s[...] = o_ref[...]


def _set2set(x, batch, wih, whh, bih, bhh):
    n = x.shape[0]
    nb = n // _NB
    full = lambda s: pl.BlockSpec(s, lambda st, p, b: (0,) * len(s))
    return pl.pallas_call(
        _set2set_body,
        grid=(5, 2, nb),
        in_specs=[
            pl.BlockSpec((_NB, 64), lambda st, p, b: (b, 0)),
            pl.BlockSpec((_NB, 1), lambda st, p, b: (b, 0)),
            full(wih.shape), full(whh.shape), full((1, 256)),
            full((1, 256)),
        ],
        out_specs=pl.BlockSpec((16, 128), lambda st, p, b: (0, 0)),
        out_shape=jax.ShapeDtypeStruct((16, 128), jnp.float32),
        scratch_shapes=[
            pltpu.VMEM((16, 64), jnp.float32),
            pltpu.VMEM((16, 64), jnp.float32),
            pltpu.VMEM((16, 128), jnp.float32),
            pltpu.VMEM((1, 16), jnp.float32),
            pltpu.VMEM((16, 128), jnp.float32),
        ],
    )(x, batch.reshape(n, 1), wih, whh, bih.reshape(1, -1),
      bhh.reshape(1, -1))


def kernel(x, edge_attr, edge_index, batch, enc_W1, enc_b1, ln_w, ln_b,
           enc_W2, enc_b2, g1_W, g1_as, g1_ad, g1_b, g2_W, g2_as, g2_ad,
           g2_b, g3_W, g3_as, g3_ad, g3_b, Wih, Whh, bih, bhh):
    n = x.shape[0]
    e_orig = edge_index.shape[1]
    e_live = e_orig + n
    e2p = ((e_live + 4095) // 4096) * 4096
    src0, dst0 = edge_index[0], edge_index[1]
    loop = jnp.arange(n, dtype=src0.dtype)
    padz = jnp.zeros((e2p - e_live,), src0.dtype)
    src = jnp.concatenate([src0, loop, padz])
    dst = jnp.concatenate([dst0, loop, padz])
    zden = jnp.zeros((6272,), jnp.float32)
    zacc = jnp.zeros((784, 32), jnp.float32)

    h = _encoder(x, enc_W1, enc_b1, ln_w, ln_b, enc_W2, enc_b2)

    for (w, a_s, a_d, b, heads, ch) in (
            (g1_W, g1_as, g1_ad, g1_b, 2, 64),
            (g2_W, g2_as, g2_ad, g2_b, 2, 64),
            (g3_W, g3_as, g3_ad, g3_b, 1, 64)):
        nslice = heads * ch // 32
        xs_list, att16, gm = _transform(h, w, a_s, a_d, heads, ch)
        gmax16 = jnp.concatenate(
            [gm[0, 0:heads] + gm[0, heads:2 * heads],
             jnp.zeros((16 - heads,), jnp.float32)])
        a_cols = ([att16[:, h] for h in range(heads)]
                  + [att16[:, heads + h] for h in range(heads)])
        ex_t, den2 = _make_edge_ex(heads, e_orig, e_live, e2p)(
            *a_cols, src, dst, gmax16, zden)
        num_parts = _make_edge_agg(nslice, e2p)(
            *xs_list, src, dst, ex_t, zacc)
        h = _normalize(num_parts, den2, b, heads, nslice)

    return _set2set(h, batch, Wih, Whh, bih, bhh)


# speedup vs baseline: 41.5270x; 1.0596x over previous
import functools
import jax
import jax.numpy as jnp
from jax.experimental import pallas as pl
from jax.experimental.pallas import tpu as pltpu
from jax.experimental.pallas import tpu_sc as plsc

_NB = 5000
_N = 50000
_NPAD = 50176
_DEN = 100352
_W = 128


def _encoder_body(x_ref, w1_ref, b1_ref, lnw_ref, lnb_ref, w2_ref, b2_ref,
                  o_ref):
    h = jnp.dot(x_ref[...], w1_ref[...].T,
                preferred_element_type=jnp.float32) + b1_ref[...]
    mu = jnp.mean(h, axis=-1, keepdims=True)
    var = jnp.mean((h - mu) ** 2, axis=-1, keepdims=True)
    h = (h - mu) * jax.lax.rsqrt(var + 1e-5) * lnw_ref[...] + lnb_ref[...]
    h = jnp.where(h >= 0, h, 0.1 * h)
    o_ref[...] = jnp.dot(h, w2_ref[...].T,
                         preferred_element_type=jnp.float32) + b2_ref[...]


def _encoder(x, w1, b1, lnw, lnb, w2, b2):
    n = x.shape[0]
    grid = n // _NB
    full = lambda s: pl.BlockSpec(s, lambda i: (0,) * len(s))
    return pl.pallas_call(
        _encoder_body,
        grid=(grid,),
        in_specs=[
            pl.BlockSpec((_NB, x.shape[1]), lambda i: (i, 0)),
            full(w1.shape), full((1, 128)), full((1, 128)), full((1, 128)),
            full(w2.shape), full((1, 256)),
        ],
        out_specs=pl.BlockSpec((_NB, 256), lambda i: (i, 0)),
        out_shape=jax.ShapeDtypeStruct((n, 256), jnp.float32),
    )(x, w1, b1.reshape(1, -1), lnw.reshape(1, -1), lnb.reshape(1, -1),
      w2, b2.reshape(1, -1))


def _transform_body(h_ref, w_ref, as_ref, ad_ref, *out_refs, heads, ch,
                    nslice):
    xs_refs = out_refs[:nslice]
    att_ref, gm_ref = out_refs[nslice], out_refs[nslice + 1]
    i = pl.program_id(0)
    xs = jnp.dot(h_ref[...], w_ref[...].T,
                 preferred_element_type=jnp.float32)
    for sl in range(nslice):
        xs_refs[sl][...] = xs[:, 32 * sl:32 * sl + 32]
    x3 = xs.reshape(xs.shape[0], heads, ch)
    a_s = jnp.sum(x3 * as_ref[...], axis=-1)
    a_d = jnp.sum(x3 * ad_ref[...], axis=-1)
    nb = xs.shape[0]
    att_ref[...] = jnp.concatenate(
        [a_s, a_d, jnp.zeros((nb, 16 - 2 * heads), jnp.float32)], axis=-1)
    bm = jnp.concatenate(
        [jnp.max(a_s, axis=0, keepdims=True),
         jnp.max(a_d, axis=0, keepdims=True),
         jnp.full((1, 128 - 2 * heads), -1e30, jnp.float32)], axis=-1)

    @pl.when(i == 0)
    def _init():
        gm_ref[...] = jnp.full_like(gm_ref, -1e30)

    gm_ref[...] = jnp.maximum(gm_ref[...], bm)


def _transform(h, w, att_s, att_d, heads, ch):
    n = h.shape[0]
    grid = n // _NB
    nslice = heads * ch // 32
    full = lambda s: pl.BlockSpec(s, lambda i: (0,) * len(s))
    outs = pl.pallas_call(
        functools.partial(_transform_body, heads=heads, ch=ch,
                          nslice=nslice),
        grid=(grid,),
        in_specs=[
            pl.BlockSpec((_NB, h.shape[1]), lambda i: (i, 0)),
            full(w.shape), full(att_s.shape), full(att_d.shape),
        ],
        out_specs=(
            [pl.BlockSpec((_NB, 32), lambda i: (i, 0))] * nslice
            + [pl.BlockSpec((_NB, 16), lambda i: (i, 0)),
               pl.BlockSpec((1, 128), lambda i: (0, 0))]),
        out_shape=(
            [jax.ShapeDtypeStruct((n, 32), jnp.float32)] * nslice
            + [jax.ShapeDtypeStruct((n, 16), jnp.float32),
               jax.ShapeDtypeStruct((1, 128), jnp.float32)]),
    )(h, w, att_s, att_d)
    return outs[:nslice], outs[nslice], outs[nslice + 1]


def _norm_body(np_ref, den_ref, b_ref, o_ref, *, heads, nslice):
    d = den_ref[...][:, 0:2] + den_ref[...][:, 2:4]
    npv = np_ref[...]
    cols = []
    for sl in range(nslice):
        num = (npv[:, 32 * sl:32 * sl + 32]
               + npv[:, 32 * (nslice + sl):32 * (nslice + sl) + 32])
        dc = d[:, sl // 2:sl // 2 + 1]
        cols.append(num / (dc + 1e-16))
    o_ref[...] = jnp.concatenate(cols, axis=-1) + b_ref[...]


def _normalize(num_parts, den2, bias, heads, nslice):
    hc = 32 * nslice
    grid = _N // _NB
    np2 = jnp.transpose(num_parts[:, :_N], (1, 0, 2)).reshape(
        _N, 2 * nslice * 32)
    den3 = jnp.transpose(den2.reshape(2, _DEN // 2, 2)[:, :_N],
                         (1, 0, 2)).reshape(_N, 4)
    full = lambda s: pl.BlockSpec(s, lambda i: (0,) * len(s))
    return pl.pallas_call(
        functools.partial(_norm_body, heads=heads, nslice=nslice),
        grid=(grid,),
        in_specs=[
            pl.BlockSpec((_NB, 2 * nslice * 32), lambda i: (i, 0)),
            pl.BlockSpec((_NB, 4), lambda i: (i, 0)),
            full((1, hc)),
        ],
        out_specs=pl.BlockSpec((_NB, hc), lambda i: (i, 0)),
        out_shape=jax.ShapeDtypeStruct((_N, hc), jnp.float32),
    )(np2, den3, bias.reshape(1, -1))


def _bc16(v, j):
    idx = jnp.full((16, 1), j, jnp.int32)
    dn = jax.lax.GatherDimensionNumbers(
        offset_dims=(), collapsed_slice_dims=(0,), start_index_map=(0,))
    return jax.lax.gather(
        v, idx, dn, (1,),
        mode=jax.lax.GatherScatterMode.PROMISE_IN_BOUNDS)


def _make_edge_ex(heads, e_orig, e_live, e2p):
    nw = e2p // (32 * _W)
    mesh = plsc.VectorSubcoreMesh(core_axis_name="c", subcore_axis_name="s")

    def body(*refs):
        as_refs = refs[:heads]
        ad_refs = refs[heads:2 * heads]
        src_ref, dst_ref, gmax_ref, zden_ref = refs[2 * heads:2 * heads + 4]
        ext_ref, den2_ref = refs[2 * heads + 4:2 * heads + 6]
        sc = refs[2 * heads + 6:]
        srcv, dstv = sc[0], sc[1]
        asb = sc[2:2 + heads]
        adb = sc[2 + heads:2 + 2 * heads]
        exst, idxb, valb, gmaxv, den_sh = sc[2 + 2 * heads:7 + 2 * heads]
        semi = sc[7 + 2 * heads:9 + 2 * heads]
        sems = sc[9 + 2 * heads:9 + 4 * heads]
        semo = sc[9 + 4 * heads:]
        c = jax.lax.axis_index("c")
        s = jax.lax.axis_index("s")
        lo0 = (c * 16 + s) * (nw * _W)
        pltpu.sync_copy(gmax_ref, gmaxv)
        pltpu.sync_copy(zden_ref, den_sh.at[pl.ds(s * 6272, 6272)])
        plsc.subcore_barrier()
        gv = gmaxv[...]
        iota = jax.lax.iota(jnp.int32, 16)

        def window(w, carry):
            off = lo0 + w * _W
            ci1 = pltpu.async_copy(src_ref.at[pl.ds(off, _W)], srcv,
                                   semi[0])
            ci2 = pltpu.async_copy(dst_ref.at[pl.ds(off, _W)], dstv,
                                   semi[1])
            ci1.wait()
            ci2.wait()
            cps = []
            for h in range(heads):
                cps.append(pltpu.async_copy(
                    as_refs[h].at[srcv], asb[h], sems[2 * h]))
                cps.append(pltpu.async_copy(
                    ad_refs[h].at[dstv], adb[h], sems[2 * h + 1]))
            for cp in cps:
                cp.wait()
            outs = []
            for h in range(heads):
                bc = _bc16(gv, h)
                for g in range(8):
                    sv = srcv[pl.ds(g * 16, 16)]
                    dv = dstv[pl.ds(g * 16, 16)]
                    rid = g * 16 + iota
                    al = asb[h][pl.ds(g * 16, 16)] + adb[h][pl.ds(g * 16, 16)]
                    al = jnp.where(al >= 0, al, 0.2 * al)
                    eid = off + rid
                    inval = (sv == dv) & ((eid < e_orig) | (eid >= e_live))
                    ex = jnp.where(inval, 0.0, jnp.exp(al - bc))
                    exst[h, pl.ds(g * 16, 16)] = ex
                    valb[h, pl.ds(g * 16, 16)] = ex
                    idxb[h, pl.ds(g * 16, 16)] = dv * 2 + h
                outs.append(pltpu.async_copy(
                    exst.at[h], ext_ref.at[h, pl.ds(off, _W)], semo[2 * h]))
                outs.append(pltpu.async_copy(
                    valb.at[h], den_sh.at[idxb.at[h]], semo[2 * h + 1],
                    add=True))
            for cp in outs:
                cp.wait()
            return carry

        jax.lax.fori_loop(0, nw, window, 0)
        plsc.subcore_barrier()
        pltpu.sync_copy(den_sh.at[pl.ds(s * 6272, 6272)],
                        den2_ref.at[c, pl.ds(s * 6272, 6272)])

    return pl.kernel(
        body,
        out_type=[jax.ShapeDtypeStruct((2, e2p), jnp.float32),
                  jax.ShapeDtypeStruct((2, _DEN), jnp.float32)],
        mesh=mesh,
        scratch_types=(
            [pltpu.VMEM((_W,), jnp.int32),
             pltpu.VMEM((_W,), jnp.int32)]
            + [pltpu.VMEM((_W,), jnp.float32)] * (2 * heads)
            + [pltpu.VMEM((2, _W), jnp.float32),
               pltpu.VMEM((2, _W), jnp.int32),
               pltpu.VMEM((2, _W), jnp.float32),
               pltpu.VMEM((16,), jnp.float32),
               pltpu.VMEM_SHARED((_DEN,), jnp.float32)]
            + [pltpu.SemaphoreType.DMA] * (2 + 4 * heads)),
    )


def _make_edge_agg(nslice, e2p):
    nw = e2p // (32 * _W)
    mesh = plsc.VectorSubcoreMesh(core_axis_name="c", subcore_axis_name="s")

    def body(*refs):
        xs_refs = refs[:nslice]
        src_ref, dst_ref, ext_ref, zacc_ref = refs[nslice:nslice + 4]
        out_ref = refs[nslice + 4]
        sc = refs[nslice + 5:]
        srcw, dstw, exw, rows, scaled = (sc[0:2], sc[2:4], sc[4:6],
                                         sc[6:8], sc[8:10])
        acc_sh = sc[10]
        sidx = sc[11:17]
        sgat = sc[17:19]
        ssc = sc[19:21]
        c = jax.lax.axis_index("c")
        s = jax.lax.axis_index("s")
        lo0 = (c * 16 + s) * (nw * _W)

        def compute(p):
            for g in range(8):
                exv = exw[p][pl.ds(g * 16, 16)]
                for j in range(16):
                    e = g * 16 + j
                    bc = _bc16(exv, j)
                    scaled[p][e, pl.ds(0, 16)] = (rows[p][e, pl.ds(0, 16)]
                                                  * bc)
                    scaled[p][e, pl.ds(16, 16)] = (rows[p][e, pl.ds(16, 16)]
                                                   * bc)

        for sl in range(nslice):
            h = sl // 2
            for k in range(4):
                pltpu.sync_copy(
                    zacc_ref, acc_sh.at[pl.ds(s * 3136 + k * 784, 784)])
            plsc.subcore_barrier()

            def pair(wp, carry):
                offs = [lo0 + (2 * wp + p) * _W for p in range(2)]
                cps = []
                for p in range(2):
                    cps.append([
                        pltpu.async_copy(src_ref.at[pl.ds(offs[p], _W)],
                                         srcw[p], sidx[3 * p]),
                        pltpu.async_copy(dst_ref.at[pl.ds(offs[p], _W)],
                                         dstw[p], sidx[3 * p + 1]),
                        pltpu.async_copy(ext_ref.at[h, pl.ds(offs[p], _W)],
                                         exw[p], sidx[3 * p + 2])])
                gth = []
                for p in range(2):
                    for cp in cps[p]:
                        cp.wait()
                    gth.append(pltpu.async_copy(
                        xs_refs[sl].at[srcw[p]], rows[p], sgat[p]))
                gth[0].wait()
                compute(0)
                sc0 = pltpu.async_copy(scaled[0], acc_sh.at[dstw[0]],
                                       ssc[0], add=True)
                gth[1].wait()
                compute(1)
                sc1 = pltpu.async_copy(scaled[1], acc_sh.at[dstw[1]],
                                       ssc[1], add=True)
                sc0.wait()
                sc1.wait()
                return carry

            jax.lax.fori_loop(0, nw // 2, pair, 0)
            plsc.subcore_barrier()
            pltpu.sync_copy(
                acc_sh.at[pl.ds(s * 3136, 3136)],
                out_ref.at[c * nslice + sl, pl.ds(s * 3136, 3136)])
            plsc.subcore_barrier()

    return pl.kernel(
        body,
        out_type=jax.ShapeDtypeStruct((2 * nslice, _NPAD, 32), jnp.float32),
        mesh=mesh,
        compiler_params=pltpu.CompilerParams(use_tc_tiling_on_sc=False),
        scratch_types=(
            [pltpu.VMEM((_W,), jnp.int32)] * 2
            + [pltpu.VMEM((_W,), jnp.int32)] * 2
            + [pltpu.VMEM((_W,), jnp.float32)] * 2
            + [pltpu.VMEM((_W, 32), jnp.float32)] * 2
            + [pltpu.VMEM((_W, 32), jnp.float32)] * 2
            + [pltpu.VMEM_SHARED((_NPAD, 32), jnp.float32)]
            + [pltpu.SemaphoreType.DMA] * 10),
    )


def _dotT(a, b):
    return jax.lax.dot_general(a, b, (((1,), (1,)), ((), ())),
                               preferred_element_type=jnp.float32)


def _dot0(a, b):
    return jax.lax.dot_general(a, b, (((0,), (0,)), ((), ())),
                               preferred_element_type=jnp.float32)


def _set2set_body(x_ref, bat_ref, wih_ref, whh_ref, bih_ref, bhh_ref, o_ref,
                  h_s, c_s, q_s, emax_s, acc_s):
    s = pl.program_id(0)
    p = pl.program_id(1)
    b = pl.program_id(2)
    nb = pl.num_programs(2)
    neg = jnp.float32(-1e30)

    @pl.when((p == 0) & (b == 0))
    def _step_head():
        @pl.when(s == 0)
        def _init():
            h_s[...] = jnp.zeros_like(h_s)
            c_s[...] = jnp.zeros_like(c_s)
            q_s[...] = jnp.zeros_like(q_s)

        g = (_dotT(q_s[...], wih_ref[...]) + bih_ref[...]
             + _dotT(h_s[...], whh_ref[...]) + bhh_ref[...])
        i = jax.nn.sigmoid(g[:, 0:64])
        f = jax.nn.sigmoid(g[:, 64:128])
        gg = jnp.tanh(g[:, 128:192])
        o = jax.nn.sigmoid(g[:, 192:256])
        c = f * c_s[...] + i * gg
        c_s[...] = c
        h_s[...] = o * jnp.tanh(c)
        emax_s[...] = jnp.full_like(emax_s, neg)

    blk = x_ref.shape[0]
    mask = (bat_ref[...] ==
            jax.lax.broadcasted_iota(jnp.int32, (blk, 16), 1))
    e = _dotT(x_ref[...], h_s[...])

    @pl.when(p == 0)
    def _max_pass():
        em = jnp.where(mask, e, neg)
        bm = jnp.max(em, axis=0, keepdims=True)
        emax_s[...] = jnp.maximum(emax_s[...], bm)

    @pl.when((p == 1) & (b == 0))
    def _fixup():
        m = emax_s[...]
        emax_s[...] = jnp.where(m > jnp.float32(-1e29), m, 0.0)
        acc_s[...] = jnp.zeros_like(acc_s)

    @pl.when(p == 1)
    def _acc_pass():
        ex = jnp.where(mask, jnp.exp(e - emax_s[...]), 0.0)
        xe = jnp.concatenate(
            [x_ref[...], jnp.ones((blk, 64), jnp.float32)], axis=-1)
        acc_s[...] += _dot0(ex, xe)

    @pl.when((p == 1) & (b == nb - 1))
    def _step_tail():
        acc = acc_s[...]
        r = acc[:, 0:64] / (acc[:, 64:65] + 1e-16)
        o_ref[...] = jnp.concatenate([h_s[...], r], axis=-1)
        q_s[...] = o_ref[...]


def _set2set(x, batch, wih, whh, bih, bhh):
    n = x.shape[0]
    nb = n // _NB
    full = lambda s: pl.BlockSpec(s, lambda st, p, b: (0,) * len(s))
    return pl.pallas_call(
        _set2set_body,
        grid=(5, 2, nb),
        in_specs=[
            pl.BlockSpec((_NB, 64), lambda st, p, b: (b, 0)),
            pl.BlockSpec((_NB, 1), lambda st, p, b: (b, 0)),
            full(wih.shape), full(whh.shape), full((1, 256)),
            full((1, 256)),
        ],
        out_specs=pl.BlockSpec((16, 128), lambda st, p, b: (0, 0)),
        out_shape=jax.ShapeDtypeStruct((16, 128), jnp.float32),
        scratch_shapes=[
            pltpu.VMEM((16, 64), jnp.float32),
            pltpu.VMEM((16, 64), jnp.float32),
            pltpu.VMEM((16, 128), jnp.float32),
            pltpu.VMEM((1, 16), jnp.float32),
            pltpu.VMEM((16, 128), jnp.float32),
        ],
    )(x, batch.reshape(n, 1), wih, whh, bih.reshape(1, -1),
      bhh.reshape(1, -1))


def kernel(x, edge_attr, edge_index, batch, enc_W1, enc_b1, ln_w, ln_b,
           enc_W2, enc_b2, g1_W, g1_as, g1_ad, g1_b, g2_W, g2_as, g2_ad,
           g2_b, g3_W, g3_as, g3_ad, g3_b, Wih, Whh, bih, bhh):
    n = x.shape[0]
    e_orig = edge_index.shape[1]
    e_live = e_orig + n
    e2p = ((e_live + 4095) // 4096) * 4096
    src0, dst0 = edge_index[0], edge_index[1]
    loop = jnp.arange(n, dtype=src0.dtype)
    padz = jnp.zeros((e2p - e_live,), src0.dtype)
    src = jnp.concatenate([src0, loop, padz])
    dst = jnp.concatenate([dst0, loop, padz])
    zden = jnp.zeros((6272,), jnp.float32)
    zacc = jnp.zeros((784, 32), jnp.float32)

    h = _encoder(x, enc_W1, enc_b1, ln_w, ln_b, enc_W2, enc_b2)

    for (w, a_s, a_d, b, heads, ch) in (
            (g1_W, g1_as, g1_ad, g1_b, 2, 64),
            (g2_W, g2_as, g2_ad, g2_b, 2, 64),
            (g3_W, g3_as, g3_ad, g3_b, 1, 64)):
        nslice = heads * ch // 32
        xs_list, att16, gm = _transform(h, w, a_s, a_d, heads, ch)
        gmax16 = jnp.concatenate(
            [gm[0, 0:heads] + gm[0, heads:2 * heads],
             jnp.zeros((16 - heads,), jnp.float32)])
        a_cols = ([att16[:, h] for h in range(heads)]
                  + [att16[:, heads + h] for h in range(heads)])
        ex_t, den2 = _make_edge_ex(heads, e_orig, e_live, e2p)(
            *a_cols, src, dst, gmax16, zden)
        num_parts = _make_edge_agg(nslice, e2p)(
            *xs_list, src, dst, ex_t, zacc)
        h = _normalize(num_parts, den2, b, heads, nslice)

    return _set2set(h, batch, Wih, Whh, bih, bhh)
